# Initial kernel scaffold; baseline (speedup 1.0000x reference)
import jax
import jax.numpy as jnp
from jax.experimental import pallas as pl


def kernel(input, indices):
    raise NotImplementedError("write your pallas kernel here")


# speedup vs baseline: 14.7471x; 14.7471x over previous
import functools

import jax
import jax.numpy as jnp
from jax import lax
from jax.experimental import pallas as pl
from jax.experimental.pallas import tpu as pltpu
from jax.experimental.pallas import tpu_sc as plsc

_VD = 23040
_NROWS = 16
_N = 128 * 128 * 90 * 2
_R = 4
_NG = _NROWS // _R
_NW = 32
_SUB_PER_G = _NW // _NG
_PER_SUB = _N // _SUB_PER_G
_K = 2048
_CHUNKS = _PER_SUB // _K
_TWO23 = 8388608.0


def _sc_gather(x_flat, idx):
    mesh = plsc.VectorSubcoreMesh(core_axis_name="c", subcore_axis_name="s")

    @functools.partial(
        pl.kernel,
        out_type=jax.ShapeDtypeStruct((_NROWS * _N,), jnp.float32),
        mesh=mesh,
        compiler_params=pltpu.CompilerParams(needs_layout_passes=False),
        scratch_types=[
            pltpu.VMEM((_R * _VD,), jnp.float32),
            pltpu.VMEM((_K,), jnp.float32),
            pltpu.VMEM((_K,), jnp.float32),
            pltpu.VMEM((_K,), jnp.float32),
            pltpu.VMEM((_K,), jnp.float32),
            pltpu.VMEM((_K,), jnp.float32),
        ],
    )
    def k(x_hbm, idx_hbm, out_hbm, tab_v, idx_v, o0, o1, o2, o3):
        outs = (o0, o1, o2, o3)
        cid = lax.axis_index("c")
        sid = lax.axis_index("s")
        wid = sid * 2 + cid
        g = wid // _SUB_PER_G
        slot = wid % _SUB_PER_G

        pltpu.sync_copy(x_hbm.at[pl.ds(g * (_R * _VD), _R * _VD)], tab_v)

        def chunk_body(c, carry):
            off = slot * _PER_SUB + c * _K
            pltpu.sync_copy(idx_hbm.at[pl.ds(off, _K)], idx_v)

            def vec_body(i, carry2):
                f = idx_v[pl.ds(i * 16, 16)]
                y = (f + _TWO23) - _TWO23
                y = jnp.minimum(y, float(_VD - 1))
                ii = y.astype(jnp.int32)
                for r in range(_R):
                    vals = plsc.load_gather(tab_v, [ii + r * _VD])
                    outs[r][pl.ds(i * 16, 16)] = vals
                return carry2

            lax.fori_loop(0, _K // 16, vec_body, 0, unroll=2)
            for r in range(_R):
                pltpu.sync_copy(
                    outs[r], out_hbm.at[pl.ds((g * _R + r) * _N + off, _K)]
                )
            return carry

        lax.fori_loop(0, _CHUNKS, chunk_body, 0)

    return k(x_flat, idx)


def kernel(input, indices):
    x_flat = input.reshape(_NROWS * _VD)
    out = _sc_gather(x_flat, indices)
    return out.reshape(input.shape[0], _NROWS // input.shape[0], -1, 180)


# speedup vs baseline: 16.7728x; 1.1374x over previous
import functools

import jax
import jax.numpy as jnp
from jax import lax
from jax.experimental import pallas as pl
from jax.experimental.pallas import tpu as pltpu
from jax.experimental.pallas import tpu_sc as plsc

_VD = 23040
_NROWS = 16
_N = 128 * 128 * 90 * 2
_R = 4
_NG = _NROWS // _R
_NW = 32
_SUB_PER_G = _NW // _NG
_PER_SUB = _N // _SUB_PER_G
_K = 2048
_CHUNKS = _PER_SUB // _K
_TWO23 = 8388608.0


def _sc_gather(x_flat, idx):
    mesh = plsc.VectorSubcoreMesh(core_axis_name="c", subcore_axis_name="s")

    @functools.partial(
        pl.kernel,
        out_type=jax.ShapeDtypeStruct((_NROWS * _N,), jnp.float32),
        mesh=mesh,
        compiler_params=pltpu.CompilerParams(needs_layout_passes=False),
        scratch_types=[
            pltpu.VMEM((_R * _VD,), jnp.float32),
            pltpu.VMEM((2, _K), jnp.float32),
            pltpu.VMEM((2, _R, _K), jnp.float32),
            pltpu.SemaphoreType.DMA,
            pltpu.SemaphoreType.DMA,
            pltpu.SemaphoreType.DMA,
            pltpu.SemaphoreType.DMA,
            pltpu.SemaphoreType.DMA,
        ],
    )
    def k(x_hbm, idx_hbm, out_hbm, tab_v, ibuf, obuf,
          s_tab, s_in0, s_in1, s_out0, s_out1):
        s_in = (s_in0, s_in1)
        s_out = (s_out0, s_out1)
        cid = lax.axis_index("c")
        sid = lax.axis_index("s")
        wid = sid * 2 + cid
        g = wid // _SUB_PER_G
        slot = wid % _SUB_PER_G
        base = slot * _PER_SUB

        tab_cp = pltpu.async_copy(
            x_hbm.at[pl.ds(g * (_R * _VD), _R * _VD)], tab_v, s_tab
        )
        pltpu.async_copy(idx_hbm.at[pl.ds(base, _K)], ibuf.at[0], s_in[0])
        tab_cp.wait()

        def pair_body(h, carry):
            for b in range(2):
                c = h * 2 + b
                off = base + c * _K

                @pl.when(c + 1 < _CHUNKS)
                def _():
                    pltpu.async_copy(
                        idx_hbm.at[pl.ds(off + _K, _K)],
                        ibuf.at[1 - b],
                        s_in[1 - b],
                    )

                pltpu.make_async_copy(
                    idx_hbm.at[pl.ds(off, _K)], ibuf.at[b], s_in[b]
                ).wait()

                @pl.when(c >= 2)
                def _():
                    for r in range(_R):
                        pltpu.make_async_copy(
                            obuf.at[b, r],
                            out_hbm.at[
                                pl.ds((g * _R + r) * _N + off - 2 * _K, _K)
                            ],
                            s_out[b],
                        ).wait()

                def vec_body(i, carry2):
                    f = ibuf[b, pl.ds(i * 16, 16)]
                    y = (f + _TWO23) - _TWO23
                    y = jnp.minimum(y, float(_VD - 1))
                    ii = y.astype(jnp.int32)
                    for r in range(_R):
                        vals = plsc.load_gather(tab_v, [ii + r * _VD])
                        obuf[b, r, pl.ds(i * 16, 16)] = vals
                    return carry2

                lax.fori_loop(0, _K // 16, vec_body, 0, unroll=4)

                for r in range(_R):
                    pltpu.async_copy(
                        obuf.at[b, r],
                        out_hbm.at[pl.ds((g * _R + r) * _N + off, _K)],
                        s_out[b],
                    )
            return carry

        lax.fori_loop(0, _CHUNKS // 2, pair_body, 0)

        for c in (_CHUNKS - 2, _CHUNKS - 1):
            b = c % 2
            off = base + c * _K
            for r in range(_R):
                pltpu.make_async_copy(
                    obuf.at[b, r],
                    out_hbm.at[pl.ds((g * _R + r) * _N + off, _K)],
                    s_out[b],
                ).wait()

    return k(x_flat, idx)


def kernel(input, indices):
    x_flat = input.reshape(_NROWS * _VD)
    out = _sc_gather(x_flat, indices)
    return out.reshape(input.shape[0], _NROWS // input.shape[0], -1, 180)


# speedup vs baseline: 29.0261x; 1.7305x over previous
import functools

import jax
import jax.numpy as jnp
from jax import lax
from jax.experimental import pallas as pl
from jax.experimental.pallas import tpu as pltpu
from jax.experimental.pallas import tpu_sc as plsc

_VD = 23040
_NROWS = 16
_N = 128 * 128 * 90 * 2
_R = 4
_NG = _NROWS // _R
_NW = 32
_SUB_PER_G = _NW // _NG
_PER_SUB = _N // _SUB_PER_G
_K = 2048
_CHUNKS = _PER_SUB // _K
_TWO23 = 8388608.0


def _sc_gather(x_flat, idx):
    mesh = plsc.VectorSubcoreMesh(core_axis_name="c", subcore_axis_name="s")

    @functools.partial(
        pl.kernel,
        out_type=jax.ShapeDtypeStruct((_NROWS * _N,), jnp.float32),
        mesh=mesh,
        compiler_params=pltpu.CompilerParams(needs_layout_passes=False),
        scratch_types=[
            pltpu.VMEM((_R * _VD,), jnp.float32),
            pltpu.VMEM((2, _K), jnp.float32),
            pltpu.VMEM((2, _R, _K), jnp.float32),
            pltpu.SemaphoreType.DMA,
            pltpu.SemaphoreType.DMA,
            pltpu.SemaphoreType.DMA,
            pltpu.SemaphoreType.DMA,
            pltpu.SemaphoreType.DMA,
        ],
    )
    def k(x_hbm, idx_hbm, out_hbm, tab_v, ibuf, obuf,
          s_tab, s_in0, s_in1, s_out0, s_out1):
        s_in = (s_in0, s_in1)
        s_out = (s_out0, s_out1)
        cid = lax.axis_index("c")
        sid = lax.axis_index("s")
        wid = sid * 2 + cid
        g = wid // _SUB_PER_G
        slot = wid % _SUB_PER_G
        base = slot * _PER_SUB

        tab_cp = pltpu.async_copy(
            x_hbm.at[pl.ds(g * (_R * _VD), _R * _VD)], tab_v, s_tab
        )
        pltpu.async_copy(idx_hbm.at[pl.ds(base, _K)], ibuf.at[0], s_in[0])
        tab_cp.wait()

        def pair_body(h, carry):
            for b in range(2):
                c = h * 2 + b
                off = base + c * _K

                @pl.when(c + 1 < _CHUNKS)
                def _():
                    pltpu.async_copy(
                        idx_hbm.at[pl.ds(off + _K, _K)],
                        ibuf.at[1 - b],
                        s_in[1 - b],
                    )

                pltpu.make_async_copy(
                    idx_hbm.at[pl.ds(off, _K)], ibuf.at[b], s_in[b]
                ).wait()

                @pl.when(c >= 2)
                def _():
                    for r in range(_R):
                        pltpu.make_async_copy(
                            obuf.at[b, r],
                            out_hbm.at[
                                pl.ds((g * _R + r) * _N + off - 2 * _K, _K)
                            ],
                            s_out[b],
                        ).wait()

                @plsc.parallel_loop(0, _K // 16, unroll=4)
                def _(i):
                    f = ibuf[b, pl.ds(i * 16, 16)]
                    zi = plsc.bitcast(f + _TWO23, jnp.int32)
                    ii = jnp.minimum(zi & 0x7FFFFF, _VD - 1)
                    for r in range(_R):
                        vals = plsc.load_gather(tab_v, [ii + r * _VD])
                        obuf[b, r, pl.ds(i * 16, 16)] = vals

                for r in range(_R):
                    pltpu.async_copy(
                        obuf.at[b, r],
                        out_hbm.at[pl.ds((g * _R + r) * _N + off, _K)],
                        s_out[b],
                    )
            return carry

        lax.fori_loop(0, _CHUNKS // 2, pair_body, 0)

        for c in (_CHUNKS - 2, _CHUNKS - 1):
            b = c % 2
            off = base + c * _K
            for r in range(_R):
                pltpu.make_async_copy(
                    obuf.at[b, r],
                    out_hbm.at[pl.ds((g * _R + r) * _N + off, _K)],
                    s_out[b],
                ).wait()

    return k(x_flat, idx)


def kernel(input, indices):
    x_flat = input.reshape(_NROWS * _VD)
    out = _sc_gather(x_flat, indices)
    return out.reshape(input.shape[0], _NROWS // input.shape[0], -1, 180)
